# baseline (speedup 1.0000x reference)
import jax
import jax.numpy as jnp
from jax import lax
from jax.experimental import pallas as pl
from jax.experimental.pallas import tpu as pltpu
from jax.experimental.pallas import tpu_sc as plsc

N = 10000
CH = 128
E = 320000
NC = 2
NS = 16
NW = NC * NS
EPT = E // NW
CHUNK = 128
NCHUNK = 80
CHG = 64
GRP = 4
TPC = 160
NGRP = TPC // GRP + 2
EPAD = NGRP * GRP * CHG
MSLOT = (TPC + GRP) * CHG
NPAD = 10240
DUMMY = N
RPT = NPAD // NS
RB = 1280

_MESH = plsc.VectorSubcoreMesh(
    core_axis_name="c", subcore_axis_name="s", num_cores=NC, num_subcores=NS
)


def _deg_body(dstp_hbm, out_hbm, dst_v, ones_v, zrow_v, deg_sh):
    c = lax.axis_index("c")
    s = lax.axis_index("s")
    w = c * NS + s
    for k in range(CHUNK // 16):
        ones_v[pl.ds(k * 16, 16)] = jnp.full((16,), 1.0, jnp.float32)
        zrow_v[pl.ds(k * 16, 16)] = jnp.zeros((16,), jnp.float32)
    for k in range(RPT // CHUNK):
        pltpu.sync_copy(zrow_v, deg_sh.at[pl.ds(s * RPT + k * CHUNK, CHUNK)])
    pltpu.sync_copy(dstp_hbm.at[w], dst_v)
    plsc.subcore_barrier()

    def body(j, carry):
        pltpu.sync_copy(ones_v, deg_sh.at[dst_v.at[j]], add=True)
        return carry

    lax.fori_loop(0, NCHUNK, body, 0)
    plsc.subcore_barrier()
    pltpu.sync_copy(deg_sh.at[pl.ds(s * RPT, RPT)], out_hbm.at[c, pl.ds(s * RPT, RPT)])


_deg_call = pl.kernel(
    _deg_body,
    out_type=jax.ShapeDtypeStruct((NC, NPAD), jnp.float32),
    mesh=_MESH,
    scratch_types=[
        pltpu.VMEM((NCHUNK, CHUNK), jnp.int32),
        pltpu.VMEM((CHUNK,), jnp.float32),
        pltpu.VMEM((CHUNK,), jnp.float32),
        pltpu.VMEM_SHARED((NPAD,), jnp.float32),
    ],
)


def _gath_body(y_hbm, srcp_hbm, out_hbm, srci_v, rows0_v, rows1_v, rows2_v,
               rows3_v, sg0, sg1, sg2, sg3, sw0, sw1, sw2, sw3, si0, si1):
    c = lax.axis_index("c")
    s = lax.axis_index("s")
    w = c * NS + s
    rows = (rows0_v, rows1_v, rows2_v, rows3_v)
    sg = (sg0, sg1, sg2, sg3)
    sw = (sw0, sw1, sw2, sw3)

    def g_start(bank, b):
        pltpu.async_copy(y_hbm.at[srci_v.at[bank, b]], rows[b], sg[b])

    def g_wait(bank, b):
        pltpu.make_async_copy(y_hbm.at[srci_v.at[bank, b]], rows[b],
                              sg[b]).wait()

    def w_start(b, o):
        pltpu.async_copy(rows[b], out_hbm.at[w, pl.ds(o, CHG)], sw[b])

    def w_wait(b, o):
        pltpu.make_async_copy(rows[b], out_hbm.at[w, pl.ds(o, CHG)],
                              sw[b]).wait()

    def i_start(bank, g, sem):
        pltpu.async_copy(srcp_hbm.at[w, g], srci_v.at[bank], sem)

    def i_wait(bank, g, sem):
        pltpu.make_async_copy(srcp_hbm.at[w, g], srci_v.at[bank], sem).wait()

    pltpu.sync_copy(srcp_hbm.at[w, 0], srci_v.at[0])
    pltpu.sync_copy(srcp_hbm.at[w, 1], srci_v.at[1])
    for b in range(GRP):
        g_start(0, b)

    def body(t, carry):
        g = 2 * t
        j0 = (2 * GRP * t) * CHG
        for b in range(GRP):
            g_wait(0, b)
            w_start(b, j0 + b * CHG)
        for b in range(GRP):
            w_wait(b, j0 + b * CHG)
            g_start(1, b)
        i_start(0, g + 2, si0)
        i_wait(0, g + 2, si0)
        for b in range(GRP):
            g_wait(1, b)
            w_start(b, j0 + (GRP + b) * CHG)
        for b in range(GRP):
            w_wait(b, j0 + (GRP + b) * CHG)
            g_start(0, b)
        i_start(1, g + 3, si1)
        i_wait(1, g + 3, si1)
        return carry

    lax.fori_loop(0, TPC // (2 * GRP), body, 0)
    for b in range(GRP):
        g_wait(0, b)


_gath_call = pl.kernel(
    _gath_body,
    out_type=jax.ShapeDtypeStruct((NW, MSLOT, CH), jnp.float32),
    mesh=_MESH,
    scratch_types=[
        pltpu.VMEM((2, GRP, CHG), jnp.int32),
        pltpu.VMEM((CHG, CH), jnp.float32),
        pltpu.VMEM((CHG, CH), jnp.float32),
        pltpu.VMEM((CHG, CH), jnp.float32),
        pltpu.VMEM((CHG, CH), jnp.float32),
        pltpu.SemaphoreType.DMA,
        pltpu.SemaphoreType.DMA,
        pltpu.SemaphoreType.DMA,
        pltpu.SemaphoreType.DMA,
        pltpu.SemaphoreType.DMA,
        pltpu.SemaphoreType.DMA,
        pltpu.SemaphoreType.DMA,
        pltpu.SemaphoreType.DMA,
        pltpu.SemaphoreType.DMA,
        pltpu.SemaphoreType.DMA,
    ],
)


def _scat_body(m_hbm, dstp_hbm, out_hbm, dsti_v, rows0_v, rows1_v, rows2_v,
               rows3_v, acc_sh, sr0, sr1, sr2, sr3, ss0, ss1, ss2, ss3,
               si0, si1):
    c = lax.axis_index("c")
    s = lax.axis_index("s")
    w = c * NS + s
    rows = (rows0_v, rows1_v, rows2_v, rows3_v)
    sr = (sr0, sr1, sr2, sr3)
    ss = (ss0, ss1, ss2, ss3)

    def r_start(b, o):
        pltpu.async_copy(m_hbm.at[w, pl.ds(o, CHG)], rows[b], sr[b])

    def r_wait(b, o):
        pltpu.make_async_copy(m_hbm.at[w, pl.ds(o, CHG)], rows[b],
                              sr[b]).wait()

    def s_start(bank, b):
        pltpu.async_copy(rows[b], acc_sh.at[dsti_v.at[bank, b]], ss[b],
                         add=True)

    def s_wait(bank, b):
        pltpu.make_async_copy(rows[b], acc_sh.at[dsti_v.at[bank, b]],
                              ss[b]).wait()

    def i_start(bank, g, sem):
        pltpu.async_copy(dstp_hbm.at[w, g], dsti_v.at[bank], sem)

    def i_wait(bank, g, sem):
        pltpu.make_async_copy(dstp_hbm.at[w, g], dsti_v.at[bank], sem).wait()

    def zr(i, carry):
        for k in range(CH // 16):
            rows0_v[i, pl.ds(k * 16, 16)] = jnp.zeros((16,), jnp.float32)
        return carry

    lax.fori_loop(0, CHG, zr, 0)
    for k in range(RPT // CHG):
        pltpu.sync_copy(rows0_v, acc_sh.at[pl.ds(s * RPT + k * CHG, CHG)])
    pltpu.sync_copy(dstp_hbm.at[w, 0], dsti_v.at[0])
    pltpu.sync_copy(dstp_hbm.at[w, 1], dsti_v.at[1])
    plsc.subcore_barrier()
    for b in range(GRP):
        r_start(b, b * CHG)

    def body(t, carry):
        g = 2 * t
        j0 = (2 * GRP * t) * CHG
        for b in range(GRP):
            r_wait(b, j0 + b * CHG)
            s_start(0, b)
        for b in range(GRP):
            s_wait(0, b)
            r_start(b, j0 + (GRP + b) * CHG)
        i_start(0, g + 2, si0)
        i_wait(0, g + 2, si0)
        for b in range(GRP):
            r_wait(b, j0 + (GRP + b) * CHG)
            s_start(1, b)
        for b in range(GRP):
            s_wait(1, b)
            r_start(b, j0 + (2 * GRP + b) * CHG)
        i_start(1, g + 3, si1)
        i_wait(1, g + 3, si1)
        return carry

    lax.fori_loop(0, TPC // (2 * GRP), body, 0)
    for b in range(GRP):
        r_wait(b, TPC * CHG + b * CHG)
    plsc.subcore_barrier()
    pltpu.sync_copy(acc_sh.at[pl.ds(s * RPT, RPT)],
                    out_hbm.at[c, pl.ds(s * RPT, RPT)])


_scat_call = pl.kernel(
    _scat_body,
    out_type=jax.ShapeDtypeStruct((NC, NPAD, CH), jnp.float32),
    mesh=_MESH,
    scratch_types=[
        pltpu.VMEM((2, GRP, CHG), jnp.int32),
        pltpu.VMEM((CHG, CH), jnp.float32),
        pltpu.VMEM((CHG, CH), jnp.float32),
        pltpu.VMEM((CHG, CH), jnp.float32),
        pltpu.VMEM((CHG, CH), jnp.float32),
        pltpu.VMEM_SHARED((NPAD, CH), jnp.float32),
        pltpu.SemaphoreType.DMA,
        pltpu.SemaphoreType.DMA,
        pltpu.SemaphoreType.DMA,
        pltpu.SemaphoreType.DMA,
        pltpu.SemaphoreType.DMA,
        pltpu.SemaphoreType.DMA,
        pltpu.SemaphoreType.DMA,
        pltpu.SemaphoreType.DMA,
        pltpu.SemaphoreType.DMA,
        pltpu.SemaphoreType.DMA,
    ],
)


def _agg_call(y, srcp, dstp):
    return _scat_call(_gath_call(y, srcp), dstp)


def _dot(a, b):
    return jnp.dot(a, b, preferred_element_type=jnp.float32,
                   precision=lax.Precision.HIGHEST)


def _tc_first_body(x_ref, w_ref, dp_ref, y_ref):
    dinv = lax.rsqrt(dp_ref[0] + dp_ref[1] + 1.0)
    y_ref[...] = _dot(x_ref[...], w_ref[...]) * dinv


def _tc_mid_body(p_ref, y_ref, dp_ref, b_ref, w_ref, o_ref):
    dinv = lax.rsqrt(dp_ref[0] + dp_ref[1] + 1.0)
    t = dinv * (p_ref[0] + p_ref[1] + y_ref[...]) + b_ref[...]
    t = jnp.maximum(t, 0.0)
    o_ref[...] = _dot(t, w_ref[...]) * dinv


def _tc_final_body(p_ref, y_ref, dp_ref, b_ref, o_ref):
    dinv = lax.rsqrt(dp_ref[0] + dp_ref[1] + 1.0)
    o_ref[...] = dinv * (p_ref[0] + p_ref[1] + y_ref[...]) + b_ref[...]


_ROWS = pl.BlockSpec((RB, CH), lambda r: (r, 0))
_WMAT = pl.BlockSpec((CH, CH), lambda r: (0, 0))
_DEGS = pl.BlockSpec((NC, RB, 1), lambda r: (0, r, 0))
_PART = pl.BlockSpec((NC, RB, CH), lambda r: (0, r, 0))
_BIAS = pl.BlockSpec((1, CH), lambda r: (0, 0))
_OUTF = jax.ShapeDtypeStruct((NPAD, CH), jnp.float32)

_tc_first = pl.pallas_call(
    _tc_first_body, grid=(NPAD // RB,),
    in_specs=[_ROWS, _WMAT, _DEGS], out_specs=_ROWS, out_shape=_OUTF)

_tc_mid = pl.pallas_call(
    _tc_mid_body, grid=(NPAD // RB,),
    in_specs=[_PART, _ROWS, _DEGS, _BIAS, _WMAT], out_specs=_ROWS,
    out_shape=_OUTF)

_tc_final = pl.pallas_call(
    _tc_final_body, grid=(NPAD // RB,),
    in_specs=[_PART, _ROWS, _DEGS, _BIAS], out_specs=_ROWS, out_shape=_OUTF)


def kernel(x, edge_index, W1, b1, W2, b2, W3, b3):
    ei = edge_index.astype(jnp.int32)
    srcp = jnp.pad(ei[0].reshape(NW, EPT), ((0, 0), (0, EPAD - EPT))
                   ).reshape(NW, NGRP, GRP, CHG)
    dstp = jnp.pad(ei[1].reshape(NW, EPT), ((0, 0), (0, EPAD - EPT)),
                   constant_values=DUMMY).reshape(NW, NGRP, GRP, CHG)
    dstd = jnp.pad(ei[1].reshape(NW, EPT), ((0, 0), (0, NCHUNK * CHUNK - EPT)),
                   constant_values=DUMMY).reshape(NW, NCHUNK, CHUNK)
    x_pad = jnp.pad(x, ((0, NPAD - N), (0, 0)))

    degp = _deg_call(dstd).reshape(NC, NPAD, 1)
    b1r, b2r, b3r = (b.reshape(1, CH) for b in (b1, b2, b3))

    y1 = _tc_first(x_pad, W1, degp)
    p1 = _agg_call(y1, srcp, dstp)
    y2 = _tc_mid(p1, y1, degp, b1r, W2)
    p2 = _agg_call(y2, srcp, dstp)
    y3 = _tc_mid(p2, y2, degp, b2r, W3)
    p3 = _agg_call(y3, srcp, dstp)
    out = _tc_final(p3, y3, degp, b3r)
    return out[:N]


# speedup vs baseline: 1.9369x; 1.9369x over previous
import jax
import jax.numpy as jnp
from jax import lax
from jax.experimental import pallas as pl
from jax.experimental.pallas import tpu as pltpu
from jax.experimental.pallas import tpu_sc as plsc

N = 10000
CH = 128
E = 320000
NC = 2
NS = 16
NW = NC * NS
EPT = E // NW
CHUNK = 128
NCHUNK = 80
NPAIR = NCHUNK // 2 + 2
EPAD = NPAIR * 2 * CHUNK
NPAD = 10240
DUMMY = N
RPT = NPAD // NS
RB = 1280

_MESH = plsc.VectorSubcoreMesh(
    core_axis_name="c", subcore_axis_name="s", num_cores=NC, num_subcores=NS
)


def _deg_body(dstp_hbm, out_hbm, dst_v, ones_v, zrow_v, deg_sh):
    c = lax.axis_index("c")
    s = lax.axis_index("s")
    w = c * NS + s
    for k in range(CHUNK // 16):
        ones_v[pl.ds(k * 16, 16)] = jnp.full((16,), 1.0, jnp.float32)
        zrow_v[pl.ds(k * 16, 16)] = jnp.zeros((16,), jnp.float32)
    for k in range(RPT // CHUNK):
        pltpu.sync_copy(zrow_v, deg_sh.at[pl.ds(s * RPT + k * CHUNK, CHUNK)])
    pltpu.sync_copy(dstp_hbm.at[w], dst_v)
    plsc.subcore_barrier()

    def body(j, carry):
        pltpu.sync_copy(ones_v, deg_sh.at[dst_v.at[j]], add=True)
        return carry

    lax.fori_loop(0, NCHUNK, body, 0)
    plsc.subcore_barrier()
    pltpu.sync_copy(deg_sh.at[pl.ds(s * RPT, RPT)], out_hbm.at[c, pl.ds(s * RPT, RPT)])


_deg_call = pl.kernel(
    _deg_body,
    out_type=jax.ShapeDtypeStruct((NC, NPAD), jnp.float32),
    mesh=_MESH,
    scratch_types=[
        pltpu.VMEM((NCHUNK, CHUNK), jnp.int32),
        pltpu.VMEM((CHUNK,), jnp.float32),
        pltpu.VMEM((CHUNK,), jnp.float32),
        pltpu.VMEM_SHARED((NPAD,), jnp.float32),
    ],
)


def _agg_body(y_hbm, srcp_hbm, dstp_hbm, out_hbm, srci_v, dsti_v, rows0_v,
              rows1_v, acc_sh, sg0, sg1, ss0, ss1, si0, si1):
    c = lax.axis_index("c")
    s = lax.axis_index("s")
    w = c * NS + s

    def g_start(bank, b, rows, sem):
        pltpu.async_copy(y_hbm.at[srci_v.at[bank, b]], rows, sem)

    def g_wait(bank, b, rows, sem):
        pltpu.make_async_copy(y_hbm.at[srci_v.at[bank, b]], rows, sem).wait()

    def s_start(bank, b, rows, sem):
        pltpu.async_copy(rows, acc_sh.at[dsti_v.at[bank, b]], sem, add=True)

    def s_wait(bank, b, rows, sem):
        pltpu.make_async_copy(rows, acc_sh.at[dsti_v.at[bank, b]], sem).wait()

    def i_start(bank, p, sem):
        pltpu.async_copy(srcp_hbm.at[w, p], srci_v.at[bank], sem)
        pltpu.async_copy(dstp_hbm.at[w, p], dsti_v.at[bank], sem)

    def i_wait(bank, p, sem):
        pltpu.make_async_copy(srcp_hbm.at[w, p], srci_v.at[bank], sem).wait()
        pltpu.make_async_copy(dstp_hbm.at[w, p], dsti_v.at[bank], sem).wait()

    def zr(i, carry):
        for k in range(CH // 16):
            rows0_v[i, pl.ds(k * 16, 16)] = jnp.zeros((16,), jnp.float32)
        return carry

    lax.fori_loop(0, CHUNK, zr, 0)
    for k in range(RPT // CHUNK):
        pltpu.sync_copy(rows0_v, acc_sh.at[pl.ds(s * RPT + k * CHUNK, CHUNK)])
    pltpu.sync_copy(srcp_hbm.at[w, 0], srci_v.at[0])
    pltpu.sync_copy(dstp_hbm.at[w, 0], dsti_v.at[0])
    pltpu.sync_copy(srcp_hbm.at[w, 1], srci_v.at[1])
    pltpu.sync_copy(dstp_hbm.at[w, 1], dsti_v.at[1])
    plsc.subcore_barrier()

    def body(t, carry):
        p = 2 * t
        d0 = pltpu.async_copy(y_hbm.at[srci_v.at[0, 0]], rows0_v, sg0)
        d1 = pltpu.async_copy(y_hbm.at[srci_v.at[0, 1]], rows1_v, sg1)
        d0.wait()
        pltpu.sync_copy(rows0_v, acc_sh.at[dsti_v.at[0, 0]], add=True)
        d2 = pltpu.async_copy(y_hbm.at[srci_v.at[1, 0]], rows0_v, sg0)
        d1.wait()
        pltpu.sync_copy(rows1_v, acc_sh.at[dsti_v.at[0, 1]], add=True)
        d3 = pltpu.async_copy(y_hbm.at[srci_v.at[1, 1]], rows1_v, sg1)
        d2.wait()
        pltpu.sync_copy(rows0_v, acc_sh.at[dsti_v.at[1, 0]], add=True)
        d3.wait()
        pltpu.sync_copy(rows1_v, acc_sh.at[dsti_v.at[1, 1]], add=True)
        i_start(0, p + 2, si0)
        i_start(1, p + 3, si1)
        i_wait(0, p + 2, si0)
        i_wait(1, p + 3, si1)
        return carry

    lax.fori_loop(0, NCHUNK // 4, body, 0)
    plsc.subcore_barrier()
    pltpu.sync_copy(acc_sh.at[pl.ds(s * RPT, RPT)],
                    out_hbm.at[c, pl.ds(s * RPT, RPT)])


_agg_call = pl.kernel(
    _agg_body,
    out_type=jax.ShapeDtypeStruct((NC, NPAD, CH), jnp.float32),
    mesh=_MESH,
    scratch_types=[
        pltpu.VMEM((2, 2, CHUNK), jnp.int32),
        pltpu.VMEM((2, 2, CHUNK), jnp.int32),
        pltpu.VMEM((CHUNK, CH), jnp.float32),
        pltpu.VMEM((CHUNK, CH), jnp.float32),
        pltpu.VMEM_SHARED((NPAD, CH), jnp.float32),
        pltpu.SemaphoreType.DMA,
        pltpu.SemaphoreType.DMA,
        pltpu.SemaphoreType.DMA,
        pltpu.SemaphoreType.DMA,
        pltpu.SemaphoreType.DMA,
        pltpu.SemaphoreType.DMA,
    ],
)


def _dot(a, b):
    return jnp.dot(a, b, preferred_element_type=jnp.float32,
                   precision=lax.Precision.HIGHEST)


def _tc_first_body(x_ref, w_ref, dp_ref, y_ref):
    dinv = lax.rsqrt(dp_ref[0] + dp_ref[1] + 1.0)
    y_ref[...] = _dot(x_ref[...], w_ref[...]) * dinv


def _tc_mid_body(p_ref, y_ref, dp_ref, b_ref, w_ref, o_ref):
    dinv = lax.rsqrt(dp_ref[0] + dp_ref[1] + 1.0)
    t = dinv * (p_ref[0] + p_ref[1] + y_ref[...]) + b_ref[...]
    t = jnp.maximum(t, 0.0)
    o_ref[...] = _dot(t, w_ref[...]) * dinv


def _tc_final_body(p_ref, y_ref, dp_ref, b_ref, o_ref):
    dinv = lax.rsqrt(dp_ref[0] + dp_ref[1] + 1.0)
    o_ref[...] = dinv * (p_ref[0] + p_ref[1] + y_ref[...]) + b_ref[...]


_ROWS = pl.BlockSpec((RB, CH), lambda r: (r, 0))
_WMAT = pl.BlockSpec((CH, CH), lambda r: (0, 0))
_DEGS = pl.BlockSpec((NC, RB, 1), lambda r: (0, r, 0))
_PART = pl.BlockSpec((NC, RB, CH), lambda r: (0, r, 0))
_BIAS = pl.BlockSpec((1, CH), lambda r: (0, 0))
_OUTF = jax.ShapeDtypeStruct((NPAD, CH), jnp.float32)

_tc_first = pl.pallas_call(
    _tc_first_body, grid=(NPAD // RB,),
    in_specs=[_ROWS, _WMAT, _DEGS], out_specs=_ROWS, out_shape=_OUTF)

_tc_mid = pl.pallas_call(
    _tc_mid_body, grid=(NPAD // RB,),
    in_specs=[_PART, _ROWS, _DEGS, _BIAS, _WMAT], out_specs=_ROWS,
    out_shape=_OUTF)

_tc_final = pl.pallas_call(
    _tc_final_body, grid=(NPAD // RB,),
    in_specs=[_PART, _ROWS, _DEGS, _BIAS], out_specs=_ROWS, out_shape=_OUTF)


def kernel(x, edge_index, W1, b1, W2, b2, W3, b3):
    ei = edge_index.astype(jnp.int32)
    srcp = jnp.pad(ei[0].reshape(NW, EPT), ((0, 0), (0, EPAD - EPT))
                   ).reshape(NW, NPAIR, 2, CHUNK)
    dstp = jnp.pad(ei[1].reshape(NW, EPT), ((0, 0), (0, EPAD - EPT)),
                   constant_values=DUMMY).reshape(NW, NPAIR, 2, CHUNK)
    dstd = dstp.reshape(NW, NPAIR * 2, CHUNK)[:, :NCHUNK]
    x_pad = jnp.pad(x, ((0, NPAD - N), (0, 0)))

    degp = _deg_call(dstd).reshape(NC, NPAD, 1)
    b1r, b2r, b3r = (b.reshape(1, CH) for b in (b1, b2, b3))

    y1 = _tc_first(x_pad, W1, degp)
    p1 = _agg_call(y1, srcp, dstp)
    y2 = _tc_mid(p1, y1, degp, b1r, W2)
    p2 = _agg_call(y2, srcp, dstp)
    y3 = _tc_mid(p2, y2, degp, b2r, W3)
    p3 = _agg_call(y3, srcp, dstp)
    out = _tc_final(p3, y3, degp, b3r)
    return out[:N]


# speedup vs baseline: 1.9435x; 1.0034x over previous
import jax
import jax.numpy as jnp
from jax import lax
from jax.experimental import pallas as pl
from jax.experimental.pallas import tpu as pltpu
from jax.experimental.pallas import tpu_sc as plsc

N = 10000
CH = 128
E = 320000
NC = 2
NS = 16
NW = NC * NS
EPT = E // NW
CHUNK = 128
NCHUNK = 80
NPAIR = NCHUNK // 2 + 2
EPAD = NPAIR * 2 * CHUNK
NPAD = 10240
DUMMY = N
RPT = NPAD // NS
RB = 1280

_MESH = plsc.VectorSubcoreMesh(
    core_axis_name="c", subcore_axis_name="s", num_cores=NC, num_subcores=NS
)


def _deg_body(dstp_hbm, out_hbm, dst_v, ones_v, zrow_v, deg_sh):
    c = lax.axis_index("c")
    s = lax.axis_index("s")
    w = c * NS + s
    for k in range(CHUNK // 16):
        ones_v[pl.ds(k * 16, 16)] = jnp.full((16,), 1.0, jnp.float32)
        zrow_v[pl.ds(k * 16, 16)] = jnp.zeros((16,), jnp.float32)
    for k in range(RPT // CHUNK):
        pltpu.sync_copy(zrow_v, deg_sh.at[pl.ds(s * RPT + k * CHUNK, CHUNK)])
    pltpu.sync_copy(dstp_hbm.at[w], dst_v)
    plsc.subcore_barrier()

    def body(j, carry):
        pltpu.sync_copy(ones_v, deg_sh.at[dst_v.at[j]], add=True)
        return carry

    lax.fori_loop(0, NCHUNK, body, 0)
    plsc.subcore_barrier()
    pltpu.sync_copy(deg_sh.at[pl.ds(s * RPT, RPT)], out_hbm.at[c, pl.ds(s * RPT, RPT)])


_deg_call = pl.kernel(
    _deg_body,
    out_type=jax.ShapeDtypeStruct((NC, NPAD), jnp.float32),
    mesh=_MESH,
    scratch_types=[
        pltpu.VMEM((NCHUNK, CHUNK), jnp.int32),
        pltpu.VMEM((CHUNK,), jnp.float32),
        pltpu.VMEM((CHUNK,), jnp.float32),
        pltpu.VMEM_SHARED((NPAD,), jnp.float32),
    ],
)


def _agg_body(y_hbm, srcp_hbm, dstp_hbm, out_hbm, srci_v, dsti_v, rows0_v,
              rows1_v, acc_sh, sg0, sg1, ss0, ss1, si0, si1):
    c = lax.axis_index("c")
    s = lax.axis_index("s")
    w = c * NS + s

    def g_start(bank, b, rows, sem):
        pltpu.async_copy(y_hbm.at[srci_v.at[bank, b]], rows, sem)

    def g_wait(bank, b, rows, sem):
        pltpu.make_async_copy(y_hbm.at[srci_v.at[bank, b]], rows, sem).wait()

    def s_start(bank, b, rows, sem):
        pltpu.async_copy(rows, acc_sh.at[dsti_v.at[bank, b]], sem, add=True)

    def s_wait(bank, b, rows, sem):
        pltpu.make_async_copy(rows, acc_sh.at[dsti_v.at[bank, b]], sem).wait()

    def i_start(bank, p, sem):
        pltpu.async_copy(srcp_hbm.at[w, p], srci_v.at[bank], sem)
        pltpu.async_copy(dstp_hbm.at[w, p], dsti_v.at[bank], sem)

    def i_wait(bank, p, sem):
        pltpu.make_async_copy(srcp_hbm.at[w, p], srci_v.at[bank], sem).wait()
        pltpu.make_async_copy(dstp_hbm.at[w, p], dsti_v.at[bank], sem).wait()

    def zr(i, carry):
        for k in range(CH // 16):
            rows0_v[i, pl.ds(k * 16, 16)] = jnp.zeros((16,), jnp.float32)
        return carry

    lax.fori_loop(0, CHUNK, zr, 0)
    for k in range(RPT // CHUNK):
        pltpu.sync_copy(rows0_v, acc_sh.at[pl.ds(s * RPT + k * CHUNK, CHUNK)])
    pltpu.sync_copy(srcp_hbm.at[w, 0], srci_v.at[0])
    pltpu.sync_copy(dstp_hbm.at[w, 0], dsti_v.at[0])
    pltpu.sync_copy(srcp_hbm.at[w, 1], srci_v.at[1])
    pltpu.sync_copy(dstp_hbm.at[w, 1], dsti_v.at[1])
    plsc.subcore_barrier()

    def body(t, carry):
        p = 2 * t
        d0 = pltpu.async_copy(y_hbm.at[srci_v.at[0, 0]], rows0_v, sg0)
        d1 = pltpu.async_copy(y_hbm.at[srci_v.at[0, 1]], rows1_v, sg1)
        d0.wait()
        pltpu.sync_copy(rows0_v, acc_sh.at[dsti_v.at[0, 0]], add=True)
        d2 = pltpu.async_copy(y_hbm.at[srci_v.at[1, 0]], rows0_v, sg0)
        d1.wait()
        pltpu.sync_copy(rows1_v, acc_sh.at[dsti_v.at[0, 1]], add=True)
        d3 = pltpu.async_copy(y_hbm.at[srci_v.at[1, 1]], rows1_v, sg1)
        d2.wait()
        pltpu.sync_copy(rows0_v, acc_sh.at[dsti_v.at[1, 0]], add=True)
        d3.wait()
        pltpu.sync_copy(rows1_v, acc_sh.at[dsti_v.at[1, 1]], add=True)
        i_start(0, p + 2, si0)
        i_start(1, p + 3, si1)
        i_wait(0, p + 2, si0)
        i_wait(1, p + 3, si1)
        return carry

    lax.fori_loop(0, NCHUNK // 4, body, 0)
    plsc.subcore_barrier()
    pltpu.sync_copy(acc_sh.at[pl.ds(s * RPT, RPT)],
                    out_hbm.at[c, pl.ds(s * RPT, RPT)])


_agg_call = pl.kernel(
    _agg_body,
    out_type=jax.ShapeDtypeStruct((NC, NPAD, CH), jnp.float32),
    mesh=_MESH,
    scratch_types=[
        pltpu.VMEM((2, 2, CHUNK), jnp.int32),
        pltpu.VMEM((2, 2, CHUNK), jnp.int32),
        pltpu.VMEM((CHUNK, CH), jnp.float32),
        pltpu.VMEM((CHUNK, CH), jnp.float32),
        pltpu.VMEM_SHARED((NPAD, CH), jnp.float32),
        pltpu.SemaphoreType.DMA,
        pltpu.SemaphoreType.DMA,
        pltpu.SemaphoreType.DMA,
        pltpu.SemaphoreType.DMA,
        pltpu.SemaphoreType.DMA,
        pltpu.SemaphoreType.DMA,
    ],
)


def _dot(a, b):
    return jnp.dot(a, b, preferred_element_type=jnp.float32)


def _tc_first_body(x_ref, w_ref, dp_ref, y_ref):
    dinv = lax.rsqrt(dp_ref[0] + dp_ref[1] + 1.0)
    y_ref[...] = _dot(x_ref[...], w_ref[...]) * dinv


def _tc_mid_body(p_ref, y_ref, dp_ref, b_ref, w_ref, o_ref):
    dinv = lax.rsqrt(dp_ref[0] + dp_ref[1] + 1.0)
    t = dinv * (p_ref[0] + p_ref[1] + y_ref[...]) + b_ref[...]
    t = jnp.maximum(t, 0.0)
    o_ref[...] = _dot(t, w_ref[...]) * dinv


def _tc_final_body(p_ref, y_ref, dp_ref, b_ref, o_ref):
    dinv = lax.rsqrt(dp_ref[0] + dp_ref[1] + 1.0)
    o_ref[...] = dinv * (p_ref[0] + p_ref[1] + y_ref[...]) + b_ref[...]


_ROWS = pl.BlockSpec((RB, CH), lambda r: (r, 0))
_WMAT = pl.BlockSpec((CH, CH), lambda r: (0, 0))
_DEGS = pl.BlockSpec((NC, RB, 1), lambda r: (0, r, 0))
_PART = pl.BlockSpec((NC, RB, CH), lambda r: (0, r, 0))
_BIAS = pl.BlockSpec((1, CH), lambda r: (0, 0))
_OUTF = jax.ShapeDtypeStruct((NPAD, CH), jnp.float32)

_tc_first = pl.pallas_call(
    _tc_first_body, grid=(NPAD // RB,),
    in_specs=[_ROWS, _WMAT, _DEGS], out_specs=_ROWS, out_shape=_OUTF)

_tc_mid = pl.pallas_call(
    _tc_mid_body, grid=(NPAD // RB,),
    in_specs=[_PART, _ROWS, _DEGS, _BIAS, _WMAT], out_specs=_ROWS,
    out_shape=_OUTF)

_tc_final = pl.pallas_call(
    _tc_final_body, grid=(NPAD // RB,),
    in_specs=[_PART, _ROWS, _DEGS, _BIAS], out_specs=_ROWS, out_shape=_OUTF)


def kernel(x, edge_index, W1, b1, W2, b2, W3, b3):
    ei = edge_index.astype(jnp.int32)
    srcp = jnp.pad(ei[0].reshape(NW, EPT), ((0, 0), (0, EPAD - EPT))
                   ).reshape(NW, NPAIR, 2, CHUNK)
    dstp = jnp.pad(ei[1].reshape(NW, EPT), ((0, 0), (0, EPAD - EPT)),
                   constant_values=DUMMY).reshape(NW, NPAIR, 2, CHUNK)
    dstd = dstp.reshape(NW, NPAIR * 2, CHUNK)[:, :NCHUNK]
    x_pad = jnp.pad(x, ((0, NPAD - N), (0, 0)))

    degp = _deg_call(dstd).reshape(NC, NPAD, 1)
    b1r, b2r, b3r = (b.reshape(1, CH) for b in (b1, b2, b3))

    y1 = _tc_first(x_pad, W1, degp)
    p1 = _agg_call(y1, srcp, dstp)
    y2 = _tc_mid(p1, y1, degp, b1r, W2)
    p2 = _agg_call(y2, srcp, dstp)
    y3 = _tc_mid(p2, y2, degp, b2r, W3)
    p3 = _agg_call(y3, srcp, dstp)
    out = _tc_final(p3, y3, degp, b3r)
    return out[:N]


# speedup vs baseline: 1.9489x; 1.0028x over previous
import jax
import jax.numpy as jnp
from jax import lax
from jax.experimental import pallas as pl
from jax.experimental.pallas import tpu as pltpu
from jax.experimental.pallas import tpu_sc as plsc

N = 10000
CH = 128
E = 320000
NC = 2
NS = 16
NW = NC * NS
EPT = E // NW
CHUNK = 128
NCHUNK = 80
NPAIR = NCHUNK // 2 + 2
EPAD = NPAIR * 2 * CHUNK
NPAD = 10240
DUMMY = N
RPT = NPAD // NS
RB = 1280

_MESH = plsc.VectorSubcoreMesh(
    core_axis_name="c", subcore_axis_name="s", num_cores=NC, num_subcores=NS
)


def _deg_body(dstp_hbm, out_hbm, dst_v, ones_v, zrow_v, deg_sh):
    c = lax.axis_index("c")
    s = lax.axis_index("s")
    w = c * NS + s
    for k in range(CHUNK // 16):
        ones_v[pl.ds(k * 16, 16)] = jnp.full((16,), 1.0, jnp.float32)
        zrow_v[pl.ds(k * 16, 16)] = jnp.zeros((16,), jnp.float32)
    for k in range(RPT // CHUNK):
        pltpu.sync_copy(zrow_v, deg_sh.at[pl.ds(s * RPT + k * CHUNK, CHUNK)])
    pltpu.sync_copy(dstp_hbm.at[w], dst_v)
    plsc.subcore_barrier()

    def body(j, carry):
        pltpu.sync_copy(ones_v, deg_sh.at[dst_v.at[j]], add=True)
        return carry

    lax.fori_loop(0, NCHUNK, body, 0)
    plsc.subcore_barrier()
    pltpu.sync_copy(deg_sh.at[pl.ds(s * RPT, RPT)], out_hbm.at[c, pl.ds(s * RPT, RPT)])


_deg_call = pl.kernel(
    _deg_body,
    out_type=jax.ShapeDtypeStruct((NC, NPAD), jnp.float32),
    mesh=_MESH,
    scratch_types=[
        pltpu.VMEM((NCHUNK, CHUNK), jnp.int32),
        pltpu.VMEM((CHUNK,), jnp.float32),
        pltpu.VMEM((CHUNK,), jnp.float32),
        pltpu.VMEM_SHARED((NPAD,), jnp.float32),
    ],
)


def _agg_body(y_hbm, srcp_hbm, dstp_hbm, out_hbm, srci_v, dsti_v, rows0_v,
              rows1_v, acc_sh, sg0, sg1, ss0, ss1, si0, si1):
    c = lax.axis_index("c")
    s = lax.axis_index("s")
    w = c * NS + s

    def g_start(bank, b, rows, sem):
        pltpu.async_copy(y_hbm.at[srci_v.at[bank, b]], rows, sem)

    def g_wait(bank, b, rows, sem):
        pltpu.make_async_copy(y_hbm.at[srci_v.at[bank, b]], rows, sem).wait()

    def s_start(bank, b, rows, sem):
        pltpu.async_copy(rows, acc_sh.at[dsti_v.at[bank, b]], sem, add=True)

    def s_wait(bank, b, rows, sem):
        pltpu.make_async_copy(rows, acc_sh.at[dsti_v.at[bank, b]], sem).wait()

    def i_start(bank, p, sem):
        pltpu.async_copy(srcp_hbm.at[w, p], srci_v.at[bank], sem)
        pltpu.async_copy(dstp_hbm.at[w, p], dsti_v.at[bank], sem)

    def i_wait(bank, p, sem):
        pltpu.make_async_copy(srcp_hbm.at[w, p], srci_v.at[bank], sem).wait()
        pltpu.make_async_copy(dstp_hbm.at[w, p], dsti_v.at[bank], sem).wait()

    def zr(i, carry):
        for k in range(CH // 16):
            rows0_v[i, pl.ds(k * 16, 16)] = jnp.zeros((16,), jnp.float32)
        return carry

    lax.fori_loop(0, CHUNK, zr, 0)
    for k in range(RPT // CHUNK):
        pltpu.sync_copy(rows0_v, acc_sh.at[pl.ds(s * RPT + k * CHUNK, CHUNK)])
    pltpu.sync_copy(srcp_hbm.at[w, 0], srci_v.at[0])
    pltpu.sync_copy(dstp_hbm.at[w, 0], dsti_v.at[0])
    pltpu.sync_copy(srcp_hbm.at[w, 1], srci_v.at[1])
    pltpu.sync_copy(dstp_hbm.at[w, 1], dsti_v.at[1])
    plsc.subcore_barrier()

    def body(t, carry):
        p = 2 * t
        d0 = pltpu.async_copy(y_hbm.at[srci_v.at[0, 0]], rows0_v, sg0)
        d1 = pltpu.async_copy(y_hbm.at[srci_v.at[0, 1]], rows1_v, sg1)
        d0.wait()
        pltpu.sync_copy(rows0_v, acc_sh.at[dsti_v.at[0, 0]], add=True)
        d2 = pltpu.async_copy(y_hbm.at[srci_v.at[1, 0]], rows0_v, sg0)
        d1.wait()
        pltpu.sync_copy(rows1_v, acc_sh.at[dsti_v.at[0, 1]], add=True)
        d3 = pltpu.async_copy(y_hbm.at[srci_v.at[1, 1]], rows1_v, sg1)
        i_start(0, p + 2, si0)
        d2.wait()
        pltpu.sync_copy(rows0_v, acc_sh.at[dsti_v.at[1, 0]], add=True)
        d3.wait()
        pltpu.sync_copy(rows1_v, acc_sh.at[dsti_v.at[1, 1]], add=True)
        i_start(1, p + 3, si1)
        i_wait(0, p + 2, si0)
        i_wait(1, p + 3, si1)
        return carry

    lax.fori_loop(0, NCHUNK // 4, body, 0)
    plsc.subcore_barrier()
    pltpu.sync_copy(acc_sh.at[pl.ds(s * RPT, RPT)],
                    out_hbm.at[c, pl.ds(s * RPT, RPT)])


_agg_call = pl.kernel(
    _agg_body,
    out_type=jax.ShapeDtypeStruct((NC, NPAD, CH), jnp.float32),
    mesh=_MESH,
    scratch_types=[
        pltpu.VMEM((2, 2, CHUNK), jnp.int32),
        pltpu.VMEM((2, 2, CHUNK), jnp.int32),
        pltpu.VMEM((CHUNK, CH), jnp.float32),
        pltpu.VMEM((CHUNK, CH), jnp.float32),
        pltpu.VMEM_SHARED((NPAD, CH), jnp.float32),
        pltpu.SemaphoreType.DMA,
        pltpu.SemaphoreType.DMA,
        pltpu.SemaphoreType.DMA,
        pltpu.SemaphoreType.DMA,
        pltpu.SemaphoreType.DMA,
        pltpu.SemaphoreType.DMA,
    ],
)


def _dot(a, b):
    return jnp.dot(a, b, preferred_element_type=jnp.float32)


def _tc_first_body(x_ref, w_ref, dp_ref, y_ref):
    dinv = lax.rsqrt(dp_ref[0] + dp_ref[1] + 1.0)
    y_ref[...] = _dot(x_ref[...], w_ref[...]) * dinv


def _tc_mid_body(p_ref, y_ref, dp_ref, b_ref, w_ref, o_ref):
    dinv = lax.rsqrt(dp_ref[0] + dp_ref[1] + 1.0)
    t = dinv * (p_ref[0] + p_ref[1] + y_ref[...]) + b_ref[...]
    t = jnp.maximum(t, 0.0)
    o_ref[...] = _dot(t, w_ref[...]) * dinv


def _tc_final_body(p_ref, y_ref, dp_ref, b_ref, o_ref):
    dinv = lax.rsqrt(dp_ref[0] + dp_ref[1] + 1.0)
    o_ref[...] = dinv * (p_ref[0] + p_ref[1] + y_ref[...]) + b_ref[...]


_ROWS = pl.BlockSpec((RB, CH), lambda r: (r, 0))
_WMAT = pl.BlockSpec((CH, CH), lambda r: (0, 0))
_DEGS = pl.BlockSpec((NC, RB, 1), lambda r: (0, r, 0))
_PART = pl.BlockSpec((NC, RB, CH), lambda r: (0, r, 0))
_BIAS = pl.BlockSpec((1, CH), lambda r: (0, 0))
_OUTF = jax.ShapeDtypeStruct((NPAD, CH), jnp.float32)

_tc_first = pl.pallas_call(
    _tc_first_body, grid=(NPAD // RB,),
    in_specs=[_ROWS, _WMAT, _DEGS], out_specs=_ROWS, out_shape=_OUTF)

_tc_mid = pl.pallas_call(
    _tc_mid_body, grid=(NPAD // RB,),
    in_specs=[_PART, _ROWS, _DEGS, _BIAS, _WMAT], out_specs=_ROWS,
    out_shape=_OUTF)

_tc_final = pl.pallas_call(
    _tc_final_body, grid=(NPAD // RB,),
    in_specs=[_PART, _ROWS, _DEGS, _BIAS], out_specs=_ROWS, out_shape=_OUTF)


def kernel(x, edge_index, W1, b1, W2, b2, W3, b3):
    ei = edge_index.astype(jnp.int32)
    srcp = jnp.pad(ei[0].reshape(NW, EPT), ((0, 0), (0, EPAD - EPT))
                   ).reshape(NW, NPAIR, 2, CHUNK)
    dstp = jnp.pad(ei[1].reshape(NW, EPT), ((0, 0), (0, EPAD - EPT)),
                   constant_values=DUMMY).reshape(NW, NPAIR, 2, CHUNK)
    dstd = dstp.reshape(NW, NPAIR * 2, CHUNK)[:, :NCHUNK]
    x_pad = jnp.pad(x, ((0, NPAD - N), (0, 0)))

    degp = _deg_call(dstd).reshape(NC, NPAD, 1)
    b1r, b2r, b3r = (b.reshape(1, CH) for b in (b1, b2, b3))

    y1 = _tc_first(x_pad, W1, degp)
    p1 = _agg_call(y1, srcp, dstp)
    y2 = _tc_mid(p1, y1, degp, b1r, W2)
    p2 = _agg_call(y2, srcp, dstp)
    y3 = _tc_mid(p2, y2, degp, b2r, W3)
    p3 = _agg_call(y3, srcp, dstp)
    out = _tc_final(p3, y3, degp, b3r)
    return out[:N]
